# baseline (speedup 1.0000x reference)
import functools

import jax
import jax.numpy as jnp
from jax import lax
from jax.experimental import pallas as pl
from jax.experimental.pallas import tpu as pltpu
from jax.experimental.pallas import tpu_sc as plsc

_B = 16384
_C = 5
_D = 64
_NW = 32
_BW = _B // _NW
_CH = 128
_NCH = _BW // _CH


def _sc_body(ctx_idx_hbm, tgt_idx_hbm, ctx_table, tgt_table, wb_hbm,
             out_hbm, idx_v, tgt_idx_v, ctx_rows, tgt_rows, out_v, wb_v,
             sem):
    wid = lax.axis_index("s") * 2 + lax.axis_index("c")
    base = wid * _BW

    pltpu.sync_copy(wb_hbm, wb_v)

    iota = lax.iota(jnp.int32, 16)
    tidx_pats = []
    wpats = []
    for v in range(20):
        k = iota + (16 * v)
        tidx_pats.append(k // 5)
        wpats.append(plsc.load_gather(wb_v, [k % 5]))

    for ch in range(_NCH):
        cb = base + ch * _CH
        pltpu.sync_copy(ctx_idx_hbm.at[cb // _CH], idx_v)
        pltpu.sync_copy(tgt_idx_hbm.at[pl.ds(cb, _CH)], tgt_idx_v)
        copies = []
        for j in range(_C):
            copies.append(pltpu.async_copy(
                ctx_table.at[idx_v.at[j]],
                ctx_rows.at[pl.ds(j * _CH, _CH)], sem))
        copies.append(pltpu.async_copy(
            tgt_table.at[tgt_idx_v], tgt_rows, sem))
        for cp in copies:
            cp.wait()

        def body(e, carry):
            full_e = jnp.full((16,), 0, jnp.int32) + e
            acc = jnp.zeros((16,), jnp.float32)
            for v in range(20):
                c_o, q = divmod(v, 4)
                cvec = ctx_rows[_C * e + c_o, pl.ds(16 * q, 16)]
                texp = plsc.load_gather(tgt_rows, [full_e, tidx_pats[v]])
                acc = acc + cvec * wpats[v] * texp
            s = jnp.zeros((16,), jnp.float32) + jnp.sum(acc)
            plsc.store_scatter(out_v, [full_e + (ch * _CH)], s,
                               mask=iota == 0)
            return carry

        lax.fori_loop(0, _CH, body, 0)

    bb = plsc.load_gather(wb_v, [jnp.full((16,), _C, jnp.int32)])
    for i in range(_BW // 16):
        x = out_v[pl.ds(16 * i, 16)] + bb
        out_v[pl.ds(16 * i, 16)] = 1.0 / (1.0 + jnp.exp(-x))
    pltpu.sync_copy(out_v, out_hbm.at[pl.ds(base, _BW)])


@jax.jit
def kernel(context_input, target_input, context_table, target_table,
           W_dense, b_dense):
    ctx_idx = context_input.reshape(_B // _CH, _C, _CH)
    tgt_idx = target_input.reshape(_B)
    wb = jnp.concatenate([W_dense.reshape(_C), b_dense,
                          jnp.zeros((2,), jnp.float32)])

    mesh = plsc.VectorSubcoreMesh(core_axis_name="c", subcore_axis_name="s",
                                  num_cores=2, num_subcores=16)
    run = pl.kernel(
        _sc_body,
        out_type=jax.ShapeDtypeStruct((_B,), jnp.float32),
        mesh=mesh,
        compiler_params=pltpu.CompilerParams(use_tc_tiling_on_sc=False,
                                             needs_layout_passes=False),
        scratch_types=[
            pltpu.VMEM((_C, _CH), jnp.int32),
            pltpu.VMEM((_CH,), jnp.int32),
            pltpu.VMEM((_C * _CH, _D), jnp.float32),
            pltpu.VMEM((_CH, _D), jnp.float32),
            pltpu.VMEM((_BW,), jnp.float32),
            pltpu.VMEM((8,), jnp.float32),
            pltpu.SemaphoreType.DMA,
        ],
    )
    out = run(ctx_idx, tgt_idx, context_table, target_table, wb)
    return out.reshape(_B, 1)
